# baseline (speedup 1.0000x reference)
import jax
import jax.numpy as jnp
import numpy as np
from jax import lax
from jax.experimental import pallas as pl
from jax.experimental.pallas import tpu as pltpu
from jax.experimental.pallas import tpu_sc as plsc

D_MODEL = 256
N_HEADS = 8
N_LEVELS = 4
N_POINTS = 4
D_HEAD = 32
SPATIAL = [(64, 64), (32, 32), (16, 16), (8, 8)]
LEVEL_START = [0, 4096, 5120, 5376]
LEN_IN = 5440
B = 4
LEN_Q = 900

NW = 32
N_TRIPLE = B * LEN_Q * N_HEADS
TPW = N_TRIPLE // NW
STAGE = 60
N_STAGE = TPW // STAGE
CHUNK = 2
NBUF = 6
N_CHUNK = STAGE // CHUNK
N_ROW = N_LEVELS * N_POINTS * 4
N_TAB = B * LEN_IN * N_HEADS
N_COL = N_HEADS * N_ROW


def _col_consts():
    wl = np.zeros((1, N_COL), np.float32)
    hl = np.zeros((1, N_COL), np.float32)
    ls8 = np.zeros((1, N_COL), np.float32)
    hc = np.zeros((1, N_COL), np.float32)
    dx = np.zeros((1, N_COL), np.float32)
    dy = np.zeros((1, N_COL), np.float32)
    for h in range(N_HEADS):
        for l in range(N_LEVELS):
            for p in range(N_POINTS):
                for cr in range(4):
                    c = h * 64 + (l * 4 + p) * 4 + cr
                    wl[0, c] = SPATIAL[l][1]
                    hl[0, c] = SPATIAL[l][0]
                    ls8[0, c] = LEVEL_START[l] * N_HEADS
                    hc[0, c] = h
                    dx[0, c] = cr & 1
                    dy[0, c] = cr >> 1
    gones = np.zeros((N_COL, N_COL), np.float32)
    for g in range(N_HEADS):
        gones[g * 64:(g + 1) * 64, g * 64:(g + 1) * 64] = 0.25
    mx = np.zeros((8, N_COL), np.float32)
    my = np.zeros((8, N_COL), np.float32)
    exx = np.zeros((256, N_COL), np.float32)
    exy = np.zeros((256, N_COL), np.float32)
    exa = np.zeros((128, N_COL), np.float32)
    for c in range(N_COL):
        l = (c % 64) // 16
        mx[2 * l, c] = 1.0
        my[2 * l + 1, c] = 1.0
        h = c // 64
        lp = (c % 64) // 4
        exx[2 * (h * 16 + lp), c] = 1.0
        exy[2 * (h * 16 + lp) + 1, c] = 1.0
        exa[h * 16 + lp, c] = 1.0
    return wl, hl, ls8, hc, dx, dy, gones, mx, my, exx, exy, exa


(_WL, _HL, _LS8, _HC, _DX, _DY, _GONES, _MX, _MY,
 _EXX, _EXY, _EXA) = _col_consts()

_PERM = np.zeros((D_MODEL,), np.int64)
for _h in range(N_HEADS):
    for _j in range(D_HEAD):
        _PERM[_h * D_HEAD + _j] = (_h * D_HEAD + _j // 2
                                   + (16 if _j % 2 else 0))


def _mm_bias_body(x_ref, w_ref, b_ref, o_ref):
    o_ref[...] = (
        jnp.dot(x_ref[...], w_ref[...], preferred_element_type=jnp.float32)
        + b_ref[...]
    ).astype(o_ref.dtype)


def _mm_bias(x, w, b, blk, out_dtype=jnp.float32):
    n, k = x.shape
    m = w.shape[1]
    return pl.pallas_call(
        _mm_bias_body,
        grid=(n // blk,),
        in_specs=[
            pl.BlockSpec((blk, k), lambda i: (i, 0)),
            pl.BlockSpec((k, m), lambda i: (0, 0)),
            pl.BlockSpec((1, m), lambda i: (0, 0)),
        ],
        out_specs=pl.BlockSpec((blk, m), lambda i: (i, 0)),
        out_shape=jax.ShapeDtypeStruct((n, m), out_dtype),
    )(x, w, b)


def _out_body(x_ref, w_ref, b_ref, o_ref):
    o_ref[...] = lax.dot_general(
        x_ref[...], w_ref[...], (((1,), (1,)), ((), ())),
        preferred_element_type=jnp.float32) + b_ref[...]


def _prep_body(q_ref, rxy_ref, xf_ref, wo_ref, wa_ref, wvp_ref,
               bo_ref, ba_ref, bvp_ref, g_ref, mx_ref, my_ref,
               exx_ref, exy_ref, exa_ref,
               wl_ref, hl_ref, ls8_ref, hc_ref, dx_ref, dy_ref,
               idx_ref, w_ref, val_ref):
    f32 = jnp.float32
    dn = (((1,), (1,)), ((), ()))
    q = q_ref[0]
    wl = wl_ref[...]
    hl = hl_ref[...]
    dx = dx_ref[...]
    dy = dy_ref[...]

    val_ref[0] = (
        lax.dot_general(xf_ref[0], wvp_ref[...], dn,
                        preferred_element_type=f32) + bvp_ref[...]
    ).astype(jnp.bfloat16)

    dn0 = (((0,), (0,)), ((), ()))
    wox = lax.dot_general(wo_ref[...], exx_ref[...], dn0,
                          preferred_element_type=f32)
    woy = lax.dot_general(wo_ref[...], exy_ref[...], dn0,
                          preferred_element_type=f32)
    wat = lax.dot_general(wa_ref[...], exa_ref[...], dn0,
                          preferred_element_type=f32)
    offx = jnp.dot(q, wox, preferred_element_type=f32) + bo_ref[0]
    offy = jnp.dot(q, woy, preferred_element_type=f32) + bo_ref[1]
    e = jnp.exp(jnp.dot(q, wat, preferred_element_type=f32) + ba_ref[...])
    gs = jnp.dot(e, g_ref[...], preferred_element_type=f32)
    aw = e / gs

    refx = jnp.dot(rxy_ref[0], mx_ref[...], preferred_element_type=f32,
                   precision=lax.Precision.HIGHEST)
    refy = jnp.dot(rxy_ref[0], my_ref[...], preferred_element_type=f32,
                   precision=lax.Precision.HIGHEST)

    x = refx * wl + offx - 0.5
    y = refy * hl + offy - 0.5
    x0 = jnp.floor(x)
    y0 = jnp.floor(y)
    fx = x - x0
    fy = y - y0

    cx = x0 + dx
    cy = y0 + dy
    valid = ((cx >= 0.0) & (cx <= wl - 1.0) & (cy >= 0.0)
             & (cy <= hl - 1.0)).astype(f32)
    cx = jnp.clip(cx, 0.0, wl - 1.0)
    cy = jnp.clip(cy, 0.0, hl - 1.0)

    base = (pl.program_id(0) * (LEN_IN * N_HEADS)).astype(f32)
    base = base + ls8_ref[...] + hc_ref[...]
    idx_ref[0] = (base + (cy * wl + cx) * float(N_HEADS)).astype(jnp.int32)

    wx = 1.0 - fx - dx * (1.0 - 2.0 * fx)
    wy = 1.0 - fy - dy * (1.0 - 2.0 * fy)
    w_ref[0] = aw * wx * wy * valid


def _prep(query, rxy8, input_flatten, w_off, w_attn, w_valp, b_off, b_attn,
          b_valp):
    bspec = lambda shp: pl.BlockSpec(shp, lambda i: (0,) * len(shp))
    outs = pl.pallas_call(
        _prep_body,
        grid=(B,),
        in_specs=[
            pl.BlockSpec((1, LEN_Q, 256), lambda i: (i, 0, 0)),
            pl.BlockSpec((1, LEN_Q, 8), lambda i: (i, 0, 0)),
            pl.BlockSpec((1, LEN_IN, 256), lambda i: (i, 0, 0)),
            bspec((256, 256)), bspec((128, 256)), bspec((256, 256)),
            bspec((2, N_COL)), bspec((1, N_COL)), bspec((1, 256)),
            bspec((N_COL, N_COL)), bspec((8, N_COL)), bspec((8, N_COL)),
            bspec((256, N_COL)), bspec((256, N_COL)), bspec((128, N_COL)),
            bspec((1, N_COL)), bspec((1, N_COL)), bspec((1, N_COL)),
            bspec((1, N_COL)), bspec((1, N_COL)), bspec((1, N_COL)),
        ],
        out_specs=[pl.BlockSpec((1, LEN_Q, N_COL), lambda i: (i, 0, 0)),
                   pl.BlockSpec((1, LEN_Q, N_COL), lambda i: (i, 0, 0)),
                   pl.BlockSpec((1, LEN_IN, 256), lambda i: (i, 0, 0))],
        out_shape=[jax.ShapeDtypeStruct((B, LEN_Q, N_COL), jnp.int32),
                   jax.ShapeDtypeStruct((B, LEN_Q, N_COL), jnp.float32),
                   jax.ShapeDtypeStruct((B, LEN_IN, 256), jnp.bfloat16)],
    )(query, rxy8, input_flatten, w_off, w_attn, w_valp,
      b_off, b_attn, b_valp,
      jnp.asarray(_GONES), jnp.asarray(_MX), jnp.asarray(_MY),
      jnp.asarray(_EXX), jnp.asarray(_EXY), jnp.asarray(_EXA),
      jnp.asarray(_WL), jnp.asarray(_HL), jnp.asarray(_LS8),
      jnp.asarray(_HC), jnp.asarray(_DX), jnp.asarray(_DY))
    return outs


def _sc_body(table_hbm, idx_hbm, w_hbm, out_hbm,
             idx_v, w_v, bufs, out_v, sems):
    wid = lax.axis_index("c") * 16 + lax.axis_index("s")

    def fire(c, b):
        pltpu.async_copy(
            table_hbm.at[idx_v.at[pl.ds(c * CHUNK * N_ROW, CHUNK * N_ROW)]],
            bufs[b], sems[b])

    def drain(b):
        pltpu.make_async_copy(
            table_hbm.at[idx_v.at[pl.ds(0, CHUNK * N_ROW)]], bufs[b],
            sems[b]).wait()

    def accum(st, c, b):
        buf = bufs[b]
        for t in range(CHUNK):
            k = c * CHUNK + t
            acc = [jnp.zeros((16,), jnp.float32) for _ in range(4)]
            for g in range(4):
                wv = w_v[k, pl.ds(g * 16, 16)]
                for j in range(16):
                    r = g * 16 + j
                    sp = wv[j]
                    lo, hi = plsc.unpack(buf[t * N_ROW + r],
                                         format=plsc.PackFormat.INTERLEAVED)
                    acc[2 * (r % 2)] = acc[2 * (r % 2)] + sp * lo
                    acc[2 * (r % 2) + 1] = acc[2 * (r % 2) + 1] + sp * hi
            out_v[st * STAGE + k, pl.ds(0, 16)] = acc[0] + acc[2]
            out_v[st * STAGE + k, pl.ds(16, 16)] = acc[1] + acc[3]

    def stage_body(st, carry):
        pltpu.sync_copy(idx_hbm.at[wid, st], idx_v)
        pltpu.sync_copy(w_hbm.at[wid, st], w_v)
        for b in range(NBUF - 1):
            fire(b, b)

        def round_body(rr, carry2):
            for b in range(NBUF):
                c = rr * NBUF + b
                drain(b)
                accum(st, c, b)

                @pl.when(c + NBUF - 1 < N_CHUNK)
                def _():
                    fire(c + NBUF - 1, (b + NBUF - 1) % NBUF)
            return carry2

        lax.fori_loop(0, N_CHUNK // NBUF, round_body, 0)
        return carry

    lax.fori_loop(0, N_STAGE, stage_body, 0)
    pltpu.sync_copy(out_v, out_hbm.at[wid])


def _sc_gather(table, idx, w):
    mesh = plsc.VectorSubcoreMesh(core_axis_name="c", subcore_axis_name="s")
    kfn = pl.kernel(
        _sc_body,
        out_type=jax.ShapeDtypeStruct((NW, TPW, D_HEAD), jnp.float32),
        mesh=mesh,
        scratch_types=[
            pltpu.VMEM((STAGE * N_ROW,), jnp.int32),
            pltpu.VMEM((STAGE, N_ROW), jnp.float32),
            [pltpu.VMEM((CHUNK * N_ROW, D_HEAD), jnp.bfloat16)
             for _ in range(NBUF)],
            pltpu.VMEM((TPW, D_HEAD), jnp.float32),
            [pltpu.SemaphoreType.DMA for _ in range(NBUF)],
        ],
        compiler_params=pltpu.CompilerParams(use_tc_tiling_on_sc=False,
                                             needs_layout_passes=False),
    )
    return kfn(table, idx, w)


def kernel(query, reference_points, input_flatten, input_spatial_shapes,
           input_level_start_index, W_off, b_off, W_attn, b_attn,
           W_val, b_val, W_out, b_out):
    perm = jnp.asarray(_PERM)

    rxy8 = reference_points.reshape(B, LEN_Q, 8)
    idx512, w512, value = _prep(
        query, rxy8, input_flatten,
        W_off, W_attn, W_val[perm],
        jnp.stack([jnp.repeat(b_off[0::2], 4), jnp.repeat(b_off[1::2], 4)]),
        jnp.repeat(b_attn, 4).reshape(1, N_COL),
        b_val[perm].reshape(1, D_MODEL))
    idx = idx512.reshape(NW, N_STAGE, STAGE * N_ROW)
    w = w512.reshape(NW, N_STAGE, STAGE, N_ROW)
    table = value.reshape(N_TAB, D_HEAD)

    attn = _sc_gather(table, idx, w)
    attn = attn.reshape(B, LEN_Q, D_MODEL)

    out = pl.pallas_call(
        _out_body,
        grid=(B * LEN_Q // 600,),
        in_specs=[
            pl.BlockSpec((600, D_MODEL), lambda i: (i, 0)),
            pl.BlockSpec((D_MODEL, D_MODEL), lambda i: (0, 0)),
            pl.BlockSpec((1, D_MODEL), lambda i: (0, 0)),
        ],
        out_specs=pl.BlockSpec((600, D_MODEL), lambda i: (i, 0)),
        out_shape=jax.ShapeDtypeStruct((B * LEN_Q, D_MODEL), jnp.float32),
    )(attn.reshape(B * LEN_Q, D_MODEL), W_out, b_out.reshape(1, D_MODEL))
    return out.reshape(B, LEN_Q, D_MODEL)


# speedup vs baseline: 1.0050x; 1.0050x over previous
import jax
import jax.numpy as jnp
import numpy as np
from jax import lax
from jax.experimental import pallas as pl
from jax.experimental.pallas import tpu as pltpu
from jax.experimental.pallas import tpu_sc as plsc

D_MODEL = 256
N_HEADS = 8
N_LEVELS = 4
N_POINTS = 4
D_HEAD = 32
SPATIAL = [(64, 64), (32, 32), (16, 16), (8, 8)]
LEVEL_START = [0, 4096, 5120, 5376]
LEN_IN = 5440
B = 4
LEN_Q = 900

NW = 32
N_TRIPLE = B * LEN_Q * N_HEADS
TPW = N_TRIPLE // NW
STAGE = 60
N_STAGE = TPW // STAGE
CHUNK = 2
NBUF = 6
N_CHUNK = STAGE // CHUNK
N_ROW = N_LEVELS * N_POINTS * 4
N_TAB = B * LEN_IN * N_HEADS
N_COL = N_HEADS * N_ROW


def _col_consts():
    wl = np.zeros((1, N_COL), np.float32)
    hl = np.zeros((1, N_COL), np.float32)
    ls8 = np.zeros((1, N_COL), np.float32)
    hc = np.zeros((1, N_COL), np.float32)
    dx = np.zeros((1, N_COL), np.float32)
    dy = np.zeros((1, N_COL), np.float32)
    for h in range(N_HEADS):
        for l in range(N_LEVELS):
            for p in range(N_POINTS):
                for cr in range(4):
                    c = h * 64 + (l * 4 + p) * 4 + cr
                    wl[0, c] = SPATIAL[l][1]
                    hl[0, c] = SPATIAL[l][0]
                    ls8[0, c] = LEVEL_START[l] * N_HEADS
                    hc[0, c] = h
                    dx[0, c] = cr & 1
                    dy[0, c] = cr >> 1
    gones = np.zeros((N_COL, N_COL), np.float32)
    for g in range(N_HEADS):
        gones[g * 64:(g + 1) * 64, g * 64:(g + 1) * 64] = 0.25
    mx = np.zeros((8, N_COL), np.float32)
    my = np.zeros((8, N_COL), np.float32)
    exx = np.zeros((256, N_COL), np.float32)
    exy = np.zeros((256, N_COL), np.float32)
    exa = np.zeros((128, N_COL), np.float32)
    for c in range(N_COL):
        l = (c % 64) // 16
        mx[2 * l, c] = 1.0
        my[2 * l + 1, c] = 1.0
        h = c // 64
        lp = (c % 64) // 4
        exx[2 * (h * 16 + lp), c] = 1.0
        exy[2 * (h * 16 + lp) + 1, c] = 1.0
        exa[h * 16 + lp, c] = 1.0
    return wl, hl, ls8, hc, dx, dy, gones, mx, my, exx, exy, exa


(_WL, _HL, _LS8, _HC, _DX, _DY, _GONES, _MX, _MY,
 _EXX, _EXY, _EXA) = _col_consts()

_PERM = np.zeros((D_MODEL,), np.int64)
for _h in range(N_HEADS):
    for _j in range(D_HEAD):
        _PERM[_h * D_HEAD + _j] = (_h * D_HEAD + _j // 2
                                   + (16 if _j % 2 else 0))


def _mm_bias_body(x_ref, w_ref, b_ref, o_ref):
    o_ref[...] = (
        jnp.dot(x_ref[...], w_ref[...], preferred_element_type=jnp.float32)
        + b_ref[...]
    ).astype(o_ref.dtype)


def _mm_bias(x, w, b, blk, out_dtype=jnp.float32):
    n, k = x.shape
    m = w.shape[1]
    return pl.pallas_call(
        _mm_bias_body,
        grid=(n // blk,),
        in_specs=[
            pl.BlockSpec((blk, k), lambda i: (i, 0)),
            pl.BlockSpec((k, m), lambda i: (0, 0)),
            pl.BlockSpec((1, m), lambda i: (0, 0)),
        ],
        out_specs=pl.BlockSpec((blk, m), lambda i: (i, 0)),
        out_shape=jax.ShapeDtypeStruct((n, m), out_dtype),
    )(x, w, b)


def _out_body(x_ref, w_ref, b_ref, o_ref):
    o_ref[...] = lax.dot_general(
        x_ref[...], w_ref[...], (((1,), (1,)), ((), ())),
        preferred_element_type=jnp.float32) + b_ref[...]


def _prep_body(q_ref, rxy_ref, xf_ref, wo_ref, wa_ref, wvp_ref,
               bo_ref, ba_ref, bvp_ref, g_ref, mx_ref, my_ref,
               exx_ref, exy_ref, exa_ref,
               wl_ref, hl_ref, ls8_ref, hc_ref, dx_ref, dy_ref,
               idx_ref, w_ref, val_ref):
    f32 = jnp.float32
    dn = (((1,), (1,)), ((), ()))
    q = q_ref[0]
    wl = wl_ref[...]
    hl = hl_ref[...]
    dx = dx_ref[...]
    dy = dy_ref[...]

    val_ref[0] = (
        lax.dot_general(xf_ref[0], wvp_ref[...], dn,
                        preferred_element_type=f32) + bvp_ref[...]
    ).astype(jnp.bfloat16)

    dn0 = (((0,), (0,)), ((), ()))
    wox = lax.dot_general(wo_ref[...], exx_ref[...], dn0,
                          preferred_element_type=f32)
    woy = lax.dot_general(wo_ref[...], exy_ref[...], dn0,
                          preferred_element_type=f32)
    wat = lax.dot_general(wa_ref[...], exa_ref[...], dn0,
                          preferred_element_type=f32)
    offx = jnp.dot(q, wox, preferred_element_type=f32) + bo_ref[0]
    offy = jnp.dot(q, woy, preferred_element_type=f32) + bo_ref[1]
    e = jnp.exp(jnp.dot(q, wat, preferred_element_type=f32) + ba_ref[...])
    gs = jnp.dot(e, g_ref[...], preferred_element_type=f32)
    aw = e / gs

    refx = jnp.dot(rxy_ref[0], mx_ref[...], preferred_element_type=f32,
                   precision=lax.Precision.HIGHEST)
    refy = jnp.dot(rxy_ref[0], my_ref[...], preferred_element_type=f32,
                   precision=lax.Precision.HIGHEST)

    x = refx * wl + offx - 0.5
    y = refy * hl + offy - 0.5
    x0 = jnp.floor(x)
    y0 = jnp.floor(y)
    fx = x - x0
    fy = y - y0

    cx = x0 + dx
    cy = y0 + dy
    valid = ((cx >= 0.0) & (cx <= wl - 1.0) & (cy >= 0.0)
             & (cy <= hl - 1.0)).astype(f32)
    cx = jnp.clip(cx, 0.0, wl - 1.0)
    cy = jnp.clip(cy, 0.0, hl - 1.0)

    base = (pl.program_id(0) * (LEN_IN * N_HEADS)).astype(f32)
    base = base + ls8_ref[...] + hc_ref[...]
    idx_ref[0] = (base + (cy * wl + cx) * float(N_HEADS)).astype(jnp.int32)

    wx = 1.0 - fx - dx * (1.0 - 2.0 * fx)
    wy = 1.0 - fy - dy * (1.0 - 2.0 * fy)
    w_ref[0] = aw * wx * wy * valid


def _prep(query, rxy8, input_flatten, w_off, w_attn, w_valp, b_off, b_attn,
          b_valp):
    bspec = lambda shp: pl.BlockSpec(shp, lambda i: (0,) * len(shp))
    outs = pl.pallas_call(
        _prep_body,
        grid=(B,),
        in_specs=[
            pl.BlockSpec((1, LEN_Q, 256), lambda i: (i, 0, 0)),
            pl.BlockSpec((1, LEN_Q, 8), lambda i: (i, 0, 0)),
            pl.BlockSpec((1, LEN_IN, 256), lambda i: (i, 0, 0)),
            bspec((256, 256)), bspec((128, 256)), bspec((256, 256)),
            bspec((2, N_COL)), bspec((1, N_COL)), bspec((1, 256)),
            bspec((N_COL, N_COL)), bspec((8, N_COL)), bspec((8, N_COL)),
            bspec((256, N_COL)), bspec((256, N_COL)), bspec((128, N_COL)),
            bspec((1, N_COL)), bspec((1, N_COL)), bspec((1, N_COL)),
            bspec((1, N_COL)), bspec((1, N_COL)), bspec((1, N_COL)),
        ],
        out_specs=[pl.BlockSpec((1, LEN_Q, N_COL), lambda i: (i, 0, 0)),
                   pl.BlockSpec((1, LEN_Q, N_COL), lambda i: (i, 0, 0)),
                   pl.BlockSpec((1, LEN_IN, 256), lambda i: (i, 0, 0))],
        out_shape=[jax.ShapeDtypeStruct((B, LEN_Q, N_COL), jnp.int32),
                   jax.ShapeDtypeStruct((B, LEN_Q, N_COL), jnp.float32),
                   jax.ShapeDtypeStruct((B, LEN_IN, 256), jnp.bfloat16)],
    )(query, rxy8, input_flatten, w_off, w_attn, w_valp,
      b_off, b_attn, b_valp,
      jnp.asarray(_GONES), jnp.asarray(_MX), jnp.asarray(_MY),
      jnp.asarray(_EXX), jnp.asarray(_EXY), jnp.asarray(_EXA),
      jnp.asarray(_WL), jnp.asarray(_HL), jnp.asarray(_LS8),
      jnp.asarray(_HC), jnp.asarray(_DX), jnp.asarray(_DY))
    return outs


def _sc_body(table_hbm, idx_hbm, w_hbm, out_hbm,
             idx_v, w_v, bufs, out_v, sems):
    wid = lax.axis_index("c") * 16 + lax.axis_index("s")

    def fire(c, b):
        pltpu.async_copy(
            table_hbm.at[idx_v.at[pl.ds(c * CHUNK * N_ROW, CHUNK * N_ROW)]],
            bufs[b], sems[b])

    def drain(b):
        pltpu.make_async_copy(
            table_hbm.at[idx_v.at[pl.ds(0, CHUNK * N_ROW)]], bufs[b],
            sems[b]).wait()

    def accum(st, c, b):
        buf = bufs[b]
        for t in range(CHUNK):
            k = c * CHUNK + t
            acc = [jnp.zeros((16,), jnp.float32) for _ in range(4)]
            for g in range(4):
                wv = w_v[k, pl.ds(g * 16, 16)]
                for j in range(16):
                    r = g * 16 + j
                    sp = wv[j]
                    lo, hi = plsc.unpack(buf[t * N_ROW + r],
                                         format=plsc.PackFormat.INTERLEAVED)
                    acc[2 * (r % 2)] = acc[2 * (r % 2)] + sp * lo
                    acc[2 * (r % 2) + 1] = acc[2 * (r % 2) + 1] + sp * hi
            out_v[st * STAGE + k, pl.ds(0, 16)] = acc[0] + acc[2]
            out_v[st * STAGE + k, pl.ds(16, 16)] = acc[1] + acc[3]

    def stage_body(st, carry):
        pltpu.sync_copy(idx_hbm.at[wid, st], idx_v)
        pltpu.sync_copy(w_hbm.at[wid, st], w_v)
        for b in range(NBUF - 1):
            fire(b, b)

        @plsc.parallel_loop(0, N_CHUNK // NBUF, unroll=1)
        def round_body(rr):
            for b in range(NBUF):
                c = rr * NBUF + b
                drain(b)
                accum(st, c, b)

                @pl.when(c + NBUF - 1 < N_CHUNK)
                def _():
                    fire(c + NBUF - 1, (b + NBUF - 1) % NBUF)

        return carry

    lax.fori_loop(0, N_STAGE, stage_body, 0)
    pltpu.sync_copy(out_v, out_hbm.at[wid])


def _sc_gather(table, idx, w):
    mesh = plsc.VectorSubcoreMesh(core_axis_name="c", subcore_axis_name="s")
    kfn = pl.kernel(
        _sc_body,
        out_type=jax.ShapeDtypeStruct((NW, TPW, D_HEAD), jnp.float32),
        mesh=mesh,
        scratch_types=[
            pltpu.VMEM((STAGE * N_ROW,), jnp.int32),
            pltpu.VMEM((STAGE, N_ROW), jnp.float32),
            [pltpu.VMEM((CHUNK * N_ROW, D_HEAD), jnp.bfloat16)
             for _ in range(NBUF)],
            pltpu.VMEM((TPW, D_HEAD), jnp.float32),
            [pltpu.SemaphoreType.DMA for _ in range(NBUF)],
        ],
        compiler_params=pltpu.CompilerParams(use_tc_tiling_on_sc=False,
                                             needs_layout_passes=False),
    )
    return kfn(table, idx, w)


def kernel(query, reference_points, input_flatten, input_spatial_shapes,
           input_level_start_index, W_off, b_off, W_attn, b_attn,
           W_val, b_val, W_out, b_out):
    perm = jnp.asarray(_PERM)

    rxy8 = reference_points.reshape(B, LEN_Q, 8)
    idx512, w512, value = _prep(
        query, rxy8, input_flatten,
        W_off, W_attn, W_val[perm],
        jnp.stack([jnp.repeat(b_off[0::2], 4), jnp.repeat(b_off[1::2], 4)]),
        jnp.repeat(b_attn, 4).reshape(1, N_COL),
        b_val[perm].reshape(1, D_MODEL))
    idx = idx512.reshape(NW, N_STAGE, STAGE * N_ROW)
    w = w512.reshape(NW, N_STAGE, STAGE, N_ROW)
    table = value.reshape(N_TAB, D_HEAD)

    attn = _sc_gather(table, idx, w)
    attn = attn.reshape(B, LEN_Q, D_MODEL)

    out = pl.pallas_call(
        _out_body,
        grid=(B * LEN_Q // 600,),
        in_specs=[
            pl.BlockSpec((600, D_MODEL), lambda i: (i, 0)),
            pl.BlockSpec((D_MODEL, D_MODEL), lambda i: (0, 0)),
            pl.BlockSpec((1, D_MODEL), lambda i: (0, 0)),
        ],
        out_specs=pl.BlockSpec((600, D_MODEL), lambda i: (i, 0)),
        out_shape=jax.ShapeDtypeStruct((B * LEN_Q, D_MODEL), jnp.float32),
    )(attn.reshape(B * LEN_Q, D_MODEL), W_out, b_out.reshape(1, D_MODEL))
    return out.reshape(B, LEN_Q, D_MODEL)


# speedup vs baseline: 1.1594x; 1.1537x over previous
import jax
import jax.numpy as jnp
import numpy as np
from jax import lax
from jax.experimental import pallas as pl
from jax.experimental.pallas import tpu as pltpu
from jax.experimental.pallas import tpu_sc as plsc

D_MODEL = 256
N_HEADS = 8
N_LEVELS = 4
N_POINTS = 4
D_HEAD = 32
SPATIAL = [(64, 64), (32, 32), (16, 16), (8, 8)]
LEVEL_START = [0, 4096, 5120, 5376]
LEN_IN = 5440
B = 4
LEN_Q = 900

NW = 32
N_TRIPLE = B * LEN_Q * N_HEADS
TPW = N_TRIPLE // NW
STAGE = 300
N_STAGE = TPW // STAGE
CHUNK = 2
NBUF = 6
N_CHUNK = STAGE // CHUNK
N_ROW = N_LEVELS * N_POINTS * 4
N_TAB = B * LEN_IN * N_HEADS
N_COL = N_HEADS * N_ROW


def _col_consts():
    wl = np.zeros((1, N_COL), np.float32)
    hl = np.zeros((1, N_COL), np.float32)
    ls8 = np.zeros((1, N_COL), np.float32)
    hc = np.zeros((1, N_COL), np.float32)
    dx = np.zeros((1, N_COL), np.float32)
    dy = np.zeros((1, N_COL), np.float32)
    for h in range(N_HEADS):
        for l in range(N_LEVELS):
            for p in range(N_POINTS):
                for cr in range(4):
                    c = h * 64 + (l * 4 + p) * 4 + cr
                    wl[0, c] = SPATIAL[l][1]
                    hl[0, c] = SPATIAL[l][0]
                    ls8[0, c] = LEVEL_START[l] * N_HEADS
                    hc[0, c] = h
                    dx[0, c] = cr & 1
                    dy[0, c] = cr >> 1
    gones = np.zeros((N_COL, N_COL), np.float32)
    for g in range(N_HEADS):
        gones[g * 64:(g + 1) * 64, g * 64:(g + 1) * 64] = 0.25
    mx = np.zeros((8, N_COL), np.float32)
    my = np.zeros((8, N_COL), np.float32)
    exx = np.zeros((256, N_COL), np.float32)
    exy = np.zeros((256, N_COL), np.float32)
    exa = np.zeros((128, N_COL), np.float32)
    for c in range(N_COL):
        l = (c % 64) // 16
        mx[2 * l, c] = 1.0
        my[2 * l + 1, c] = 1.0
        h = c // 64
        lp = (c % 64) // 4
        exx[2 * (h * 16 + lp), c] = 1.0
        exy[2 * (h * 16 + lp) + 1, c] = 1.0
        exa[h * 16 + lp, c] = 1.0
    return wl, hl, ls8, hc, dx, dy, gones, mx, my, exx, exy, exa


(_WL, _HL, _LS8, _HC, _DX, _DY, _GONES, _MX, _MY,
 _EXX, _EXY, _EXA) = _col_consts()

_PERM = np.zeros((D_MODEL,), np.int64)
for _h in range(N_HEADS):
    for _j in range(D_HEAD):
        _PERM[_h * D_HEAD + _j] = (_h * D_HEAD + _j // 2
                                   + (16 if _j % 2 else 0))


def _out_body(x_ref, w_ref, b_ref, o_ref):
    o_ref[...] = lax.dot_general(
        x_ref[...], w_ref[...], (((1,), (1,)), ((), ())),
        preferred_element_type=jnp.float32) + b_ref[...]


def _prep_body(q_ref, rxy_ref, xf_ref, wo_ref, wa_ref, wvp_ref,
               bo_ref, ba_ref, bvp_ref, g_ref, mx_ref, my_ref,
               exx_ref, exy_ref, exa_ref,
               wl_ref, hl_ref, ls8_ref, hc_ref, dx_ref, dy_ref,
               idx_ref, w_ref, val_ref):
    f32 = jnp.float32
    dn = (((1,), (1,)), ((), ()))
    q = q_ref[0]
    wl = wl_ref[...]
    hl = hl_ref[...]
    dx = dx_ref[...]
    dy = dy_ref[...]

    val_ref[0] = (
        lax.dot_general(xf_ref[0], wvp_ref[...], dn,
                        preferred_element_type=f32) + bvp_ref[...]
    ).astype(jnp.bfloat16)

    dn0 = (((0,), (0,)), ((), ()))
    wox = lax.dot_general(wo_ref[...], exx_ref[...], dn0,
                          preferred_element_type=f32)
    woy = lax.dot_general(wo_ref[...], exy_ref[...], dn0,
                          preferred_element_type=f32)
    wat = lax.dot_general(wa_ref[...], exa_ref[...], dn0,
                          preferred_element_type=f32)
    offx = jnp.dot(q, wox, preferred_element_type=f32) + bo_ref[0]
    offy = jnp.dot(q, woy, preferred_element_type=f32) + bo_ref[1]
    e = jnp.exp(jnp.dot(q, wat, preferred_element_type=f32) + ba_ref[...])
    gs = jnp.dot(e, g_ref[...], preferred_element_type=f32)
    aw = e / gs

    refx = jnp.dot(rxy_ref[0], mx_ref[...], preferred_element_type=f32,
                   precision=lax.Precision.HIGHEST)
    refy = jnp.dot(rxy_ref[0], my_ref[...], preferred_element_type=f32,
                   precision=lax.Precision.HIGHEST)

    x = refx * wl + offx - 0.5
    y = refy * hl + offy - 0.5
    x0 = jnp.floor(x)
    y0 = jnp.floor(y)
    fx = x - x0
    fy = y - y0

    cx = x0 + dx
    cy = y0 + dy
    valid = ((cx >= 0.0) & (cx <= wl - 1.0) & (cy >= 0.0)
             & (cy <= hl - 1.0)).astype(f32)
    cx = jnp.clip(cx, 0.0, wl - 1.0)
    cy = jnp.clip(cy, 0.0, hl - 1.0)

    base = (pl.program_id(0) * (LEN_IN * N_HEADS)).astype(f32)
    base = base + ls8_ref[...] + hc_ref[...]
    idx_ref[0] = (base + (cy * wl + cx) * float(N_HEADS)).astype(jnp.int32)

    wx = 1.0 - fx - dx * (1.0 - 2.0 * fx)
    wy = 1.0 - fy - dy * (1.0 - 2.0 * fy)
    w_ref[0] = aw * wx * wy * valid


def _prep(query, rxy8, input_flatten, w_off, w_attn, w_valp, b_off, b_attn,
          b_valp):
    bspec = lambda shp: pl.BlockSpec(shp, lambda i: (0,) * len(shp))
    outs = pl.pallas_call(
        _prep_body,
        grid=(B,),
        in_specs=[
            pl.BlockSpec((1, LEN_Q, 256), lambda i: (i, 0, 0)),
            pl.BlockSpec((1, LEN_Q, 8), lambda i: (i, 0, 0)),
            pl.BlockSpec((1, LEN_IN, 256), lambda i: (i, 0, 0)),
            bspec((256, 256)), bspec((128, 256)), bspec((256, 256)),
            bspec((2, N_COL)), bspec((1, N_COL)), bspec((1, 256)),
            bspec((N_COL, N_COL)), bspec((8, N_COL)), bspec((8, N_COL)),
            bspec((256, N_COL)), bspec((256, N_COL)), bspec((128, N_COL)),
            bspec((1, N_COL)), bspec((1, N_COL)), bspec((1, N_COL)),
            bspec((1, N_COL)), bspec((1, N_COL)), bspec((1, N_COL)),
        ],
        out_specs=[pl.BlockSpec((1, LEN_Q, N_COL), lambda i: (i, 0, 0)),
                   pl.BlockSpec((1, LEN_Q, N_COL), lambda i: (i, 0, 0)),
                   pl.BlockSpec((1, LEN_IN, 256), lambda i: (i, 0, 0))],
        out_shape=[jax.ShapeDtypeStruct((B, LEN_Q, N_COL), jnp.int32),
                   jax.ShapeDtypeStruct((B, LEN_Q, N_COL), jnp.float32),
                   jax.ShapeDtypeStruct((B, LEN_IN, 256), jnp.bfloat16)],
    )(query, rxy8, input_flatten, w_off, w_attn, w_valp,
      b_off, b_attn, b_valp,
      jnp.asarray(_GONES), jnp.asarray(_MX), jnp.asarray(_MY),
      jnp.asarray(_EXX), jnp.asarray(_EXY), jnp.asarray(_EXA),
      jnp.asarray(_WL), jnp.asarray(_HL), jnp.asarray(_LS8),
      jnp.asarray(_HC), jnp.asarray(_DX), jnp.asarray(_DY))
    return outs


def _sc_body(table_hbm, idx_hbm, w_hbm, out_hbm,
             idx_v, w_v, bufs, out_v, sems):
    wid = lax.axis_index("c") * 16 + lax.axis_index("s")

    def fire(c, b):
        pltpu.async_copy(
            table_hbm.at[idx_v.at[pl.ds(c * CHUNK * N_ROW, CHUNK * N_ROW)]],
            bufs[b], sems[b])

    def drain(b):
        pltpu.make_async_copy(
            table_hbm.at[idx_v.at[pl.ds(0, CHUNK * N_ROW)]], bufs[b],
            sems[b]).wait()

    def accum(st, c, b):
        buf = bufs[b]
        for t in range(CHUNK):
            k = c * CHUNK + t
            acc = [jnp.zeros((16,), jnp.float32) for _ in range(4)]
            for g in range(4):
                wv = w_v[k, pl.ds(g * 16, 16)]
                for j in range(16):
                    r = g * 16 + j
                    sp = wv[j]
                    lo, hi = plsc.unpack(buf[t * N_ROW + r],
                                         format=plsc.PackFormat.INTERLEAVED)
                    acc[2 * (r % 2)] = acc[2 * (r % 2)] + sp * lo
                    acc[2 * (r % 2) + 1] = acc[2 * (r % 2) + 1] + sp * hi
            out_v[st * STAGE + k, pl.ds(0, 16)] = acc[0] + acc[2]
            out_v[st * STAGE + k, pl.ds(16, 16)] = acc[1] + acc[3]

    def stage_body(st, carry):
        pltpu.sync_copy(idx_hbm.at[wid, st], idx_v)
        pltpu.sync_copy(w_hbm.at[wid, st], w_v)
        for b in range(NBUF - 1):
            fire(b, b)

        @plsc.parallel_loop(0, N_CHUNK // NBUF, unroll=1)
        def round_body(rr):
            for b in range(NBUF):
                c = rr * NBUF + b
                drain(b)
                accum(st, c, b)

                @pl.when(c + NBUF - 1 < N_CHUNK)
                def _():
                    fire(c + NBUF - 1, (b + NBUF - 1) % NBUF)

        return carry

    lax.fori_loop(0, N_STAGE, stage_body, 0)
    pltpu.sync_copy(out_v, out_hbm.at[wid])


def _sc_gather(table, idx, w):
    mesh = plsc.VectorSubcoreMesh(core_axis_name="c", subcore_axis_name="s")
    kfn = pl.kernel(
        _sc_body,
        out_type=jax.ShapeDtypeStruct((NW, TPW, D_HEAD), jnp.float32),
        mesh=mesh,
        scratch_types=[
            pltpu.VMEM((STAGE * N_ROW,), jnp.int32),
            pltpu.VMEM((STAGE, N_ROW), jnp.float32),
            [pltpu.VMEM((CHUNK * N_ROW, D_HEAD), jnp.bfloat16)
             for _ in range(NBUF)],
            pltpu.VMEM((TPW, D_HEAD), jnp.float32),
            [pltpu.SemaphoreType.DMA for _ in range(NBUF)],
        ],
        compiler_params=pltpu.CompilerParams(use_tc_tiling_on_sc=False,
                                             needs_layout_passes=False),
    )
    return kfn(table, idx, w)


def kernel(query, reference_points, input_flatten, input_spatial_shapes,
           input_level_start_index, W_off, b_off, W_attn, b_attn,
           W_val, b_val, W_out, b_out):
    perm = jnp.asarray(_PERM)

    rxy8 = reference_points.reshape(B, LEN_Q, 8)
    idx512, w512, value = _prep(
        query, rxy8, input_flatten,
        W_off, W_attn, W_val[perm],
        jnp.stack([jnp.repeat(b_off[0::2], 4), jnp.repeat(b_off[1::2], 4)]),
        jnp.repeat(b_attn, 4).reshape(1, N_COL),
        b_val[perm].reshape(1, D_MODEL))
    idx = idx512.reshape(NW, N_STAGE, STAGE * N_ROW)
    w = w512.reshape(NW, N_STAGE, STAGE, N_ROW)
    table = value.reshape(N_TAB, D_HEAD)

    attn = _sc_gather(table, idx, w)
    attn = attn.reshape(B, LEN_Q, D_MODEL)

    out = pl.pallas_call(
        _out_body,
        grid=(B * LEN_Q // 600,),
        in_specs=[
            pl.BlockSpec((600, D_MODEL), lambda i: (i, 0)),
            pl.BlockSpec((D_MODEL, D_MODEL), lambda i: (0, 0)),
            pl.BlockSpec((1, D_MODEL), lambda i: (0, 0)),
        ],
        out_specs=pl.BlockSpec((600, D_MODEL), lambda i: (i, 0)),
        out_shape=jax.ShapeDtypeStruct((B * LEN_Q, D_MODEL), jnp.float32),
    )(attn.reshape(B * LEN_Q, D_MODEL), W_out, b_out.reshape(1, D_MODEL))
    return out.reshape(B, LEN_Q, D_MODEL)
